# baseline (speedup 1.0000x reference)
import jax
import jax.numpy as jnp
from jax.experimental import pallas as pl
from jax.experimental.pallas import tpu as pltpu

_HIDDEN = 768
_NUM_MOD = 64
_N_TOK = 2048
_TILE = 128
_NUM_TILES = _N_TOK // _TILE
_MAX_PAIRS = _NUM_TILES + _NUM_MOD - 1


def _gmm_kernel(meta_ref, x_ref, w_ref, nw_ref, out_ref):
    j = pl.program_id(1)
    tile = meta_ref[j, 0]
    row_lo = meta_ref[j, 2]
    row_hi = meta_ref[j, 3]
    prev_tile = meta_ref[jnp.maximum(j - 1, 0), 0]
    first = jnp.logical_or(j == 0, tile != prev_tile)

    normed = (x_ref[...] * (nw_ref[0] + 1.0)).astype(jnp.bfloat16)
    y = jax.lax.dot_general(
        normed,
        w_ref[0],
        dimension_numbers=(((1,), (1,)), ((), ())),
        preferred_element_type=jnp.float32,
    ).astype(jnp.bfloat16).astype(jnp.float32)

    rows = jax.lax.broadcasted_iota(jnp.int32, (_TILE, 1), 0)
    mask = jnp.logical_and(rows >= row_lo, rows < row_hi)

    @pl.when(first)
    def _():
        out_ref[...] = jnp.where(mask, y, 0.0)

    @pl.when(jnp.logical_not(first))
    def _():
        out_ref[...] = jnp.where(mask, y, out_ref[...])


def _build_meta(mm):
    e_ids = jnp.arange(_NUM_MOD, dtype=jnp.int32)
    starts = jnp.searchsorted(mm, e_ids, side="left").astype(jnp.int32)
    ends = jnp.searchsorted(mm, e_ids, side="right").astype(jnp.int32)
    first_e = mm[:: _TILE]
    last_e = mm[_TILE - 1 :: _TILE]
    off = jnp.cumsum(last_e - first_e + 1)
    j = jnp.arange(_MAX_PAIRS, dtype=jnp.int32)
    t_j = jnp.minimum(
        jnp.searchsorted(off, j, side="right").astype(jnp.int32), _NUM_TILES - 1
    )
    prev_off = jnp.where(t_j > 0, off[t_j - 1], 0).astype(jnp.int32)
    e_j = jnp.clip(first_e[t_j] + (j - prev_off), first_e[t_j], last_e[t_j])
    row_lo = jnp.clip(starts[e_j] - t_j * _TILE, 0, _TILE)
    row_hi = jnp.clip(ends[e_j] - t_j * _TILE, 0, _TILE)
    return jnp.stack([t_j, e_j, row_lo, row_hi], axis=1)


def kernel(x, modality_mapping, W, norm_w):
    mm = modality_mapping.astype(jnp.int32)
    meta = _build_meta(mm)

    nh = _HIDDEN // 2
    grid_spec = pltpu.PrefetchScalarGridSpec(
        num_scalar_prefetch=1,
        grid=(2, _MAX_PAIRS),
        in_specs=[
            pl.BlockSpec((_TILE, _HIDDEN), lambda n, j, m: (m[j, 0], 0)),
            pl.BlockSpec((1, nh, _HIDDEN), lambda n, j, m: (m[j, 1], n, 0)),
            pl.BlockSpec((1, 1, _HIDDEN), lambda n, j, m: (m[j, 1], 0, 0)),
        ],
        out_specs=pl.BlockSpec((_TILE, nh), lambda n, j, m: (m[j, 0], n)),
    )
    return pl.pallas_call(
        _gmm_kernel,
        grid_spec=grid_spec,
        out_shape=jax.ShapeDtypeStruct((_N_TOK, _HIDDEN), jnp.float32),
        compiler_params=pltpu.CompilerParams(
            dimension_semantics=("parallel", "arbitrary")
        ),
    )(meta, x, W, norm_w.reshape(_NUM_MOD, 1, _HIDDEN))


# speedup vs baseline: 1.8517x; 1.8517x over previous
import jax
import jax.numpy as jnp
from jax.experimental import pallas as pl
from jax.experimental.pallas import tpu as pltpu

_HIDDEN = 768
_NUM_MOD = 64
_N_TOK = 2048
_TILE = 128
_NUM_TILES = _N_TOK // _TILE
_MAX_PAIRS = _NUM_TILES + _NUM_MOD - 1


def _gmm_kernel(meta_ref, x_ref, w_ref, nw_ref, out_ref):
    j = pl.program_id(0)
    tile = meta_ref[0, j]
    row_lo = meta_ref[2, j]
    row_hi = meta_ref[3, j]
    prev_tile = meta_ref[0, jnp.maximum(j - 1, 0)]
    first = jnp.logical_or(j == 0, tile != prev_tile)

    normed = (x_ref[...] * (nw_ref[0] + 1.0)).astype(jnp.bfloat16)
    y = jax.lax.dot_general(
        normed,
        w_ref[0],
        dimension_numbers=(((1,), (1,)), ((), ())),
        preferred_element_type=jnp.float32,
    ).astype(jnp.bfloat16).astype(jnp.float32)

    rows = jax.lax.broadcasted_iota(jnp.int32, (_TILE, 1), 0)
    mask = jnp.logical_and(rows >= row_lo, rows < row_hi)

    @pl.when(first)
    def _():
        out_ref[...] = jnp.where(mask, y, 0.0)

    @pl.when(jnp.logical_not(first))
    def _():
        out_ref[...] = jnp.where(mask, y, out_ref[...])


def _build_meta(mm):
    e_ids = jnp.arange(_NUM_MOD, dtype=jnp.int32)
    ends = jnp.sum(mm[None, :] <= e_ids[:, None], axis=1).astype(jnp.int32)
    starts = jnp.sum(mm[None, :] < e_ids[:, None], axis=1).astype(jnp.int32)
    first_e = mm[:: _TILE]
    last_e = mm[_TILE - 1 :: _TILE]
    off = jnp.cumsum(last_e - first_e + 1).astype(jnp.int32)
    j = jnp.arange(_MAX_PAIRS, dtype=jnp.int32)
    t_j = jnp.minimum(
        jnp.sum(off[None, :] <= j[:, None], axis=1).astype(jnp.int32),
        _NUM_TILES - 1,
    )
    prev_off = jnp.where(t_j > 0, off[jnp.maximum(t_j - 1, 0)], 0).astype(jnp.int32)
    e_j = jnp.clip(first_e[t_j] + (j - prev_off), first_e[t_j], last_e[t_j])
    row_lo = jnp.clip(starts[e_j] - t_j * _TILE, 0, _TILE)
    row_hi = jnp.clip(ends[e_j] - t_j * _TILE, 0, _TILE)
    return jnp.stack([t_j, e_j, row_lo, row_hi], axis=0)


def kernel(x, modality_mapping, W, norm_w):
    mm = modality_mapping.astype(jnp.int32)
    meta = _build_meta(mm)

    grid_spec = pltpu.PrefetchScalarGridSpec(
        num_scalar_prefetch=1,
        grid=(_MAX_PAIRS,),
        in_specs=[
            pl.BlockSpec((_TILE, _HIDDEN), lambda j, m: (m[0, j], 0)),
            pl.BlockSpec((1, _HIDDEN, _HIDDEN), lambda j, m: (m[1, j], 0, 0)),
            pl.BlockSpec((1, 1, _HIDDEN), lambda j, m: (m[1, j], 0, 0)),
        ],
        out_specs=pl.BlockSpec((_TILE, _HIDDEN), lambda j, m: (m[0, j], 0)),
    )
    return pl.pallas_call(
        _gmm_kernel,
        grid_spec=grid_spec,
        out_shape=jax.ShapeDtypeStruct((_N_TOK, _HIDDEN), jnp.float32),
        compiler_params=pltpu.CompilerParams(
            dimension_semantics=("arbitrary",)
        ),
    )(meta, x, W, norm_w.reshape(_NUM_MOD, 1, _HIDDEN))
